# Initial kernel scaffold; baseline (speedup 1.0000x reference)
import functools

import jax
import jax.numpy as jnp
from jax import lax
from jax.experimental import pallas as pl
from jax.experimental.pallas import tpu as pltpu
from jax.experimental.pallas import tpu_sc as plsc

N = 100000
E = 1600000
G = 2000
B = 1000

NQ = 4
Q = 25000
ZCH = 200
SLAB_ROWS = 25200
NW = 32
CAP = 20480
ECH = 2000
STG = 4096
FLUSH = 2048
GC = 128


def _mesh():
    return plsc.VectorSubcoreMesh(core_axis_name="c", subcore_axis_name="s")


def _bucketize(edge_index):
    @functools.partial(
        pl.kernel,
        out_type=[
            jax.ShapeDtypeStruct((NQ, NW, CAP), jnp.int32),
            jax.ShapeDtypeStruct((NQ, NW, CAP), jnp.int32),
            jax.ShapeDtypeStruct((NW, 16), jnp.int32),
        ],
        mesh=_mesh(),
        scratch_types=[
            pltpu.VMEM((2, ECH), jnp.int32),
            pltpu.VMEM((NQ, STG), jnp.int32),
            pltpu.VMEM((NQ, STG), jnp.int32),
            pltpu.VMEM((16,), jnp.int32),
        ],
    )
    def kern(ei_hbm, bsrc_hbm, bdstl_hbm, counts_hbm, estg, sstg, dstg, crow):
        c = lax.axis_index("c")
        s = lax.axis_index("s")
        w = c * 16 + s
        base = w * (E // NW)
        zeros16 = jnp.zeros((16,), jnp.int32)
        trash16 = jnp.zeros((16,), jnp.int32) + (Q + w)

        def outer(i, carry):
            pltpu.sync_copy(ei_hbm.at[0, pl.ds(base + i * ECH, ECH)], estg.at[0])
            pltpu.sync_copy(ei_hbm.at[1, pl.ds(base + i * ECH, ECH)], estg.at[1])

            def vloop(v, carry2):
                offs, fls = carry2
                src16 = estg[0, pl.ds(v * 16, 16)]
                dst16 = estg[1, pl.ds(v * 16, 16)]
                q16 = (
                    (dst16 >= Q).astype(jnp.int32)
                    + (dst16 >= 2 * Q).astype(jnp.int32)
                    + (dst16 >= 3 * Q).astype(jnp.int32)
                )
                new_offs = []
                new_fls = []
                for b in range(NQ):
                    off_b = offs[b]
                    fl_b = fls[b]
                    m = q16 == b
                    plsc.store_compressed(
                        sstg.at[b, pl.ds(off_b, 16)], src16, mask=m)
                    plsc.store_compressed(
                        dstg.at[b, pl.ds(off_b, 16)], dst16 - b * Q, mask=m)
                    off_b = off_b + jnp.sum(m.astype(jnp.int32))
                    do_flush = off_b >= FLUSH

                    @pl.when(do_flush)
                    def _():
                        pltpu.sync_copy(
                            sstg.at[b, pl.ds(0, FLUSH)],
                            bsrc_hbm.at[b, w, pl.ds(fl_b, FLUSH)])
                        pltpu.sync_copy(
                            dstg.at[b, pl.ds(0, FLUSH)],
                            bdstl_hbm.at[b, w, pl.ds(fl_b, FLUSH)])
                        tail_s = sstg[b, pl.ds(FLUSH, 16)]
                        tail_d = dstg[b, pl.ds(FLUSH, 16)]
                        sstg[b, pl.ds(0, 16)] = tail_s
                        dstg[b, pl.ds(0, 16)] = tail_d

                    new_offs.append(jnp.where(do_flush, off_b - FLUSH, off_b))
                    new_fls.append(jnp.where(do_flush, fl_b + FLUSH, fl_b))
                return (new_offs, new_fls)

            return lax.fori_loop(0, ECH // 16, vloop, carry)

        zero = jnp.int32(0)
        offs, fls = lax.fori_loop(
            0, (E // NW) // ECH, outer,
            ([zero] * NQ, [zero] * NQ))

        for b in range(NQ):
            off_b = offs[b]
            fl_b = fls[b]
            total = fl_b + off_b
            padded = ((total + 127) >> 7) << 7
            for j in range(8):
                sstg[b, pl.ds(off_b + j * 16, 16)] = zeros16
                dstg[b, pl.ds(off_b + j * 16, 16)] = trash16
            pltpu.sync_copy(sstg.at[b], bsrc_hbm.at[b, w, pl.ds(fl_b, STG)])
            pltpu.sync_copy(dstg.at[b], bdstl_hbm.at[b, w, pl.ds(fl_b, STG)])
            crow[b] = padded >> 7
        for b in range(NQ, 16):
            crow[b] = jnp.int32(0)
        pltpu.sync_copy(crow, counts_hbm.at[w])

    return kern(edge_index)


def _make_aggregate(D):
    @functools.partial(
        pl.kernel,
        out_type=jax.ShapeDtypeStruct((N, D), jnp.float32),
        mesh=_mesh(),
        scratch_types=[
            pltpu.VMEM_SHARED((SLAB_ROWS, D), jnp.float32),
            pltpu.VMEM((ZCH, D), jnp.float32),
            pltpu.VMEM((GC,), jnp.int32),
            pltpu.VMEM((1, GC), jnp.int32),
            pltpu.VMEM((GC, D), jnp.float32),
            pltpu.VMEM((16,), jnp.int32),
            pltpu.SemaphoreType.DMA,
        ],
    )
    def kern(h_hbm, bsrc_hbm, bdstl_hbm, counts_hbm, agg_hbm,
             slab, zbuf, srcb, dstlb, rows, crow, sem):
        c = lax.axis_index("c")
        s = lax.axis_index("s")
        zv = jnp.zeros((16,), jnp.float32)

        def zrow(i, _):
            for j in range(D // 16):
                zbuf[i, pl.ds(j * 16, 16)] = zv
            return 0

        lax.fori_loop(0, ZCH, zrow, 0)

        for jq in range(2):
            q = 2 * c + jq
            for k in range(8):
                kk = s + k * 16

                @pl.when(kk < SLAB_ROWS // ZCH)
                def _():
                    pltpu.sync_copy(zbuf, slab.at[pl.ds(kk * ZCH, ZCH)])

            plsc.subcore_barrier()

            for jl in range(2):
                ts = s * 2 + jl
                pltpu.sync_copy(counts_hbm.at[ts], crow)
                n128 = jnp.where(c == 0, crow[jq], crow[2 + jq])

                def chunk(k, _):
                    pltpu.sync_copy(
                        bsrc_hbm.at[q, ts, pl.ds(k * GC, GC)], srcb)
                    pltpu.sync_copy(
                        bdstl_hbm.at[q, ts, pl.ds(k * GC, GC)], dstlb.at[0])
                    pltpu.async_copy(h_hbm.at[srcb], rows, sem).wait()
                    pltpu.sync_copy(rows, slab.at[dstlb.at[0]], add=True)
                    return 0

                lax.fori_loop(0, n128, chunk, 0)

            plsc.subcore_barrier()
            for k in range(8):
                kk = s + k * 16

                @pl.when(kk < Q // ZCH)
                def _():
                    pltpu.sync_copy(
                        slab.at[pl.ds(kk * ZCH, ZCH)],
                        agg_hbm.at[pl.ds(q * Q + kk * ZCH, ZCH)])

            plsc.subcore_barrier()

    return kern


_aggregate = {d: _make_aggregate(d) for d in (32, 64)}


def _layer_a(h, agg, Wa, ba):
    n, din = h.shape
    dout = Wa.shape[1]
    R = 1000

    def body(h_ref, a_ref, w_ref, b_ref, y_ref, st_ref):
        i = pl.program_id(0)
        z = jnp.dot(h_ref[...] + a_ref[...], w_ref[...],
                    preferred_element_type=jnp.float32) + b_ref[...]
        y_ref[...] = z
        sums = jnp.concatenate(
            [jnp.sum(z, axis=0, keepdims=True),
             jnp.sum(z * z, axis=0, keepdims=True)], axis=0)
        st_ref[...] = jnp.where(i == 0, sums, st_ref[...] + sums)

    return pl.pallas_call(
        body,
        grid=(n // R,),
        in_specs=[
            pl.BlockSpec((R, din), lambda i: (i, 0)),
            pl.BlockSpec((R, din), lambda i: (i, 0)),
            pl.BlockSpec((din, dout), lambda i: (0, 0)),
            pl.BlockSpec((1, dout), lambda i: (0, 0)),
        ],
        out_specs=[
            pl.BlockSpec((R, dout), lambda i: (i, 0)),
            pl.BlockSpec((2, dout), lambda i: (0, 0)),
        ],
        out_shape=[
            jax.ShapeDtypeStruct((n, dout), jnp.float32),
            jax.ShapeDtypeStruct((2, dout), jnp.float32),
        ],
    )(h, agg, Wa, ba.reshape(1, dout))


def _layer_b(y, st, g, be, Wb, bb):
    n, d = y.shape
    R = 1000

    def body(y_ref, st_ref, g_ref, be_ref, w_ref, b_ref, o_ref):
        stv = st_ref[...]
        mean = stv[0:1, :] * (1.0 / n)
        var = stv[1:2, :] * (1.0 / n) - mean * mean
        inv = g_ref[...] * lax.rsqrt(var + 1e-5)
        t = be_ref[...] - mean * inv
        z = jnp.maximum(y_ref[...] * inv + t, 0.0)
        o_ref[...] = jnp.maximum(
            jnp.dot(z, w_ref[...], preferred_element_type=jnp.float32)
            + b_ref[...], 0.0)

    return pl.pallas_call(
        body,
        grid=(n // R,),
        in_specs=[
            pl.BlockSpec((R, d), lambda i: (i, 0)),
            pl.BlockSpec((2, d), lambda i: (0, 0)),
            pl.BlockSpec((1, d), lambda i: (0, 0)),
            pl.BlockSpec((1, d), lambda i: (0, 0)),
            pl.BlockSpec((d, d), lambda i: (0, 0)),
            pl.BlockSpec((1, d), lambda i: (0, 0)),
        ],
        out_specs=pl.BlockSpec((R, d), lambda i: (i, 0)),
        out_shape=jax.ShapeDtypeStruct((n, d), jnp.float32),
    )(y, st, g.reshape(1, d), be.reshape(1, d), Wb, bb.reshape(1, d))


def _pool_nodes(h3, orders):
    HALF = N // 2
    OCH = 1000

    @functools.partial(
        pl.kernel,
        out_type=jax.ShapeDtypeStruct((2, G, 64), jnp.float32),
        mesh=_mesh(),
        scratch_types=[
            pltpu.VMEM_SHARED((G, 64), jnp.float32),
            pltpu.VMEM((125, 64), jnp.float32),
            pltpu.VMEM((OCH,), jnp.int32),
            pltpu.VMEM((1, 125), jnp.int32),
            pltpu.VMEM((OCH, 64), jnp.float32),
        ],
    )
    def kern(h_hbm, ord_hbm, out_hbm, slab, zbuf, ordb, ordrow, rowsb):
        c = lax.axis_index("c")
        s = lax.axis_index("s")
        zv = jnp.zeros((16,), jnp.float32)

        def zrow(i, _):
            for j in range(4):
                zbuf[i, pl.ds(j * 16, 16)] = zv
            return 0

        lax.fori_loop(0, 125, zrow, 0)
        pltpu.sync_copy(zbuf, slab.at[pl.ds(s * 125, 125)])
        plsc.subcore_barrier()

        for k in range(4):
            kk = s + k * 16

            @pl.when(kk < HALF // OCH)
            def _():
                base = c * HALF + kk * OCH
                pltpu.sync_copy(ord_hbm.at[pl.ds(base, OCH)], ordb)
                pltpu.sync_copy(h_hbm.at[pl.ds(base, OCH)], rowsb)
                for j in range(OCH // 125):
                    pltpu.sync_copy(ordb.at[pl.ds(j * 125, 125)], ordrow.at[0])
                    pltpu.sync_copy(rowsb.at[pl.ds(j * 125, 125)],
                                    slab.at[ordrow.at[0]], add=True)

        plsc.subcore_barrier()
        pltpu.sync_copy(slab.at[pl.ds(s * 125, 125)],
                        out_hbm.at[c, pl.ds(s * 125, 125)])

    return kern(h3, orders)


def _pool_graphs(hg, orders):
    HALF = G // 2

    @functools.partial(
        pl.kernel,
        out_type=jax.ShapeDtypeStruct((2, B, 128), jnp.float32),
        mesh=_mesh(),
        scratch_types=[
            pltpu.VMEM_SHARED((B, 128), jnp.float32),
            pltpu.VMEM((125, 128), jnp.float32),
            pltpu.VMEM((HALF,), jnp.int32),
            pltpu.VMEM((1, 125), jnp.int32),
            pltpu.VMEM((125, 128), jnp.float32),
        ],
    )
    def kern(h_hbm, ord_hbm, out_hbm, slab, zbuf, ordb, ordrow, rowsb):
        c = lax.axis_index("c")
        s = lax.axis_index("s")
        zv = jnp.zeros((16,), jnp.float32)

        def zrow(i, _):
            for j in range(8):
                zbuf[i, pl.ds(j * 16, 16)] = zv
            return 0

        lax.fori_loop(0, 125, zrow, 0)

        @pl.when(s < 8)
        def _():
            pltpu.sync_copy(zbuf, slab.at[pl.ds(s * 125, 125)])

        plsc.subcore_barrier()

        @pl.when(s < 8)
        def _():
            pltpu.sync_copy(ord_hbm.at[pl.ds(c * HALF, HALF)], ordb)
            pltpu.sync_copy(ordb.at[pl.ds(s * 125, 125)], ordrow.at[0])
            pltpu.sync_copy(h_hbm.at[pl.ds(c * HALF + s * 125, 125)], rowsb)
            pltpu.sync_copy(rowsb, slab.at[ordrow.at[0]], add=True)

        plsc.subcore_barrier()

        @pl.when(s < 8)
        def _():
            pltpu.sync_copy(slab.at[pl.ds(s * 125, 125)],
                            out_hbm.at[c, pl.ds(s * 125, 125)])

    return kern(hg, orders)


def _head1(gp, Wl1, bl1):
    def body(p_ref, w_ref, b_ref, o_ref):
        ph = p_ref[0] + p_ref[1]
        o_ref[...] = jnp.maximum(
            jnp.dot(ph, w_ref[...], preferred_element_type=jnp.float32)
            + b_ref[...], 0.0)

    return pl.pallas_call(
        body,
        out_shape=jax.ShapeDtypeStruct((G, 128), jnp.float32),
    )(gp, Wl1, bl1.reshape(1, 128))


def _head2(bp, Wl2, bl2):
    def body(p_ref, w_ref, b_ref, o_ref):
        ph = p_ref[0] + p_ref[1]
        logits = jnp.sum(ph * w_ref[...], axis=1, keepdims=True) + b_ref[...]
        o_ref[...] = 1.0 / (1.0 + jnp.exp(-logits))

    return pl.pallas_call(
        body,
        out_shape=jax.ShapeDtypeStruct((B, 1), jnp.float32),
    )(bp, Wl2.reshape(1, 128), bl2.reshape(1, 1))


def kernel(x, edge_index, nodes_orders, batch_orders,
           W1a, b1a, g1, be1, W1b, b1b,
           W2a, b2a, g2, be2, W2b, b2b,
           W3a, b3a, g3, be3, W3b, b3b,
           Wl1, bl1, Wl2, bl2):
    ei = edge_index.astype(jnp.int32)
    x32 = jnp.pad(x, ((0, 0), (0, 2)))
    W1a32 = jnp.pad(W1a, ((0, 2), (0, 0)))

    bsrc, bdstl, counts = _bucketize(ei)

    h = x32
    for Wa, ba, g, be, Wb, bb in (
        (W1a32, b1a, g1, be1, W1b, b1b),
        (W2a, b2a, g2, be2, W2b, b2b),
        (W3a, b3a, g3, be3, W3b, b3b),
    ):
        agg = _aggregate[h.shape[1]](h, bsrc, bdstl, counts)
        y, st = _layer_a(h, agg, Wa, ba)
        h = _layer_b(y, st, g, be, Wb, bb)

    gp = _pool_nodes(h, nodes_orders.astype(jnp.int32))
    hg = _head1(gp, Wl1, bl1)
    bp = _pool_graphs(hg, batch_orders.astype(jnp.int32))
    out = _head2(bp, Wl2, bl2)
    return out[:, 0]


# speedup vs baseline: 5.5786x; 5.5786x over previous
import functools

import jax
import jax.numpy as jnp
from jax import lax
from jax.experimental import pallas as pl
from jax.experimental.pallas import tpu as pltpu
from jax.experimental.pallas import tpu_sc as plsc

N = 100000
E = 1600000
G = 2000
B = 1000

NQ = 4
Q = 25000
ZCH = 200
SLAB_ROWS = 25200
NW = 32
CAP = 20480
ECH = 2000
STG = 4096
FLUSH = 2048
GC = 128


def _mesh():
    return plsc.VectorSubcoreMesh(core_axis_name="c", subcore_axis_name="s")


def _bucketize(src_e, dst_e):
    @functools.partial(
        pl.kernel,
        out_type=[
            jax.ShapeDtypeStruct((NQ * NW * CAP,), jnp.int32),
            jax.ShapeDtypeStruct((NQ * NW * CAP,), jnp.int32),
            jax.ShapeDtypeStruct((NW * 16,), jnp.int32),
        ],
        mesh=_mesh(),
        compiler_params=pltpu.CompilerParams(
            needs_layout_passes=False, use_tc_tiling_on_sc=False),
        scratch_types=[
            pltpu.VMEM((ECH,), jnp.int32),
            pltpu.VMEM((ECH,), jnp.int32),
            pltpu.VMEM((NQ * STG,), jnp.int32),
            pltpu.VMEM((NQ * STG,), jnp.int32),
            pltpu.VMEM((16,), jnp.int32),
        ],
    )
    def kern(src_hbm, dst_hbm, bsrc_hbm, bdstl_hbm, counts_hbm,
             estg0, estg1, sstg, dstg, crow):
        c = lax.axis_index("c")
        s = lax.axis_index("s")
        w = c * 16 + s
        base = w * (E // NW)
        lane = lax.iota(jnp.int32, 16)
        zeros16 = jnp.zeros((16,), jnp.int32)
        trash16 = jnp.zeros((16,), jnp.int32) + (Q + w)

        def outer(i, carry):
            eo = pl.multiple_of(base + i * ECH, 2000)
            pltpu.sync_copy(src_hbm.at[pl.ds(eo, ECH)], estg0)
            pltpu.sync_copy(dst_hbm.at[pl.ds(eo, ECH)], estg1)

            def vloop(v, carry2):
                offs, fls = carry2
                src16 = estg0[pl.ds(v * 16, 16)]
                dst16 = estg1[pl.ds(v * 16, 16)]
                q16 = (
                    (dst16 >= Q).astype(jnp.int32)
                    + (dst16 >= 2 * Q).astype(jnp.int32)
                    + (dst16 >= 3 * Q).astype(jnp.int32)
                )
                new_offs = []
                new_fls = []
                for b in range(NQ):
                    off_b = offs[b]
                    fl_b = fls[b]
                    m = q16 == b
                    cs = plsc.cumsum(m.astype(jnp.int32))
                    idx = b * STG + off_b + cs - 1
                    plsc.store_scatter(sstg, [idx], src16, mask=m)
                    plsc.store_scatter(dstg, [idx], dst16 - b * Q, mask=m)
                    off_b = off_b + cs[15]
                    do_flush = off_b >= FLUSH

                    @pl.when(do_flush)
                    def _():
                        fo = pl.multiple_of(
                            (b * NW + w) * CAP + fl_b, 2048)
                        pltpu.sync_copy(
                            sstg.at[pl.ds(b * STG, FLUSH)],
                            bsrc_hbm.at[pl.ds(fo, FLUSH)])
                        pltpu.sync_copy(
                            dstg.at[pl.ds(b * STG, FLUSH)],
                            bdstl_hbm.at[pl.ds(fo, FLUSH)])
                        tail_s = sstg[pl.ds(b * STG + FLUSH, 16)]
                        tail_d = dstg[pl.ds(b * STG + FLUSH, 16)]
                        sstg[pl.ds(b * STG, 16)] = tail_s
                        dstg[pl.ds(b * STG, 16)] = tail_d

                    new_offs.append(jnp.where(do_flush, off_b - FLUSH, off_b))
                    new_fls.append(jnp.where(do_flush, fl_b + FLUSH, fl_b))
                return (new_offs, new_fls)

            return lax.fori_loop(0, ECH // 16, vloop, carry)

        zero = jnp.int32(0)
        offs, fls = lax.fori_loop(
            0, (E // NW) // ECH, outer,
            ([zero] * NQ, [zero] * NQ))

        crow_v = jnp.zeros((16,), jnp.int32)
        for b in range(NQ):
            off_b = offs[b]
            fl_b = fls[b]
            total = fl_b + off_b
            padded = ((total + 127) >> 7) << 7
            for j in range(8):
                pidx = b * STG + off_b + j * 16 + lane
                plsc.store_scatter(sstg, [pidx], zeros16)
                plsc.store_scatter(dstg, [pidx], trash16)
            fo = pl.multiple_of((b * NW + w) * CAP + fl_b, 2048)
            pltpu.sync_copy(sstg.at[pl.ds(b * STG, STG)],
                            bsrc_hbm.at[pl.ds(fo, STG)])
            pltpu.sync_copy(dstg.at[pl.ds(b * STG, STG)],
                            bdstl_hbm.at[pl.ds(fo, STG)])
            crow_v = jnp.where(lane == b, padded >> 7, crow_v)
        crow[...] = crow_v
        pltpu.sync_copy(crow, counts_hbm.at[pl.ds(pl.multiple_of(w * 16, 16),
                                                  16)])

    return kern(src_e, dst_e)


def _make_aggregate(D):
    @functools.partial(
        pl.kernel,
        out_type=jax.ShapeDtypeStruct((N, D), jnp.float32),
        mesh=_mesh(),
        compiler_params=pltpu.CompilerParams(
            needs_layout_passes=False, use_tc_tiling_on_sc=False),
        scratch_types=[
            pltpu.VMEM_SHARED((SLAB_ROWS, D), jnp.float32),
            pltpu.VMEM((ZCH, D), jnp.float32),
            pltpu.VMEM((GC,), jnp.int32),
            pltpu.VMEM((1, GC), jnp.int32),
            pltpu.VMEM((GC, D), jnp.float32),
            pltpu.VMEM((16,), jnp.int32),
            pltpu.SemaphoreType.DMA,
        ],
    )
    def kern(h_hbm, bsrc_hbm, bdstl_hbm, counts_hbm, agg_hbm,
             slab, zbuf, srcb, dstlb, rows, crow, sem):
        c = lax.axis_index("c")
        s = lax.axis_index("s")
        zv = jnp.zeros((16,), jnp.float32)

        def zrow(i, _):
            for j in range(D // 16):
                zbuf[i, pl.ds(j * 16, 16)] = zv
            return 0

        lax.fori_loop(0, ZCH, zrow, 0)

        for jq in range(2):
            q = 2 * c + jq
            for k in range(8):
                kk = s + k * 16

                @pl.when(kk < SLAB_ROWS // ZCH)
                def _():
                    pltpu.sync_copy(
                        zbuf,
                        slab.at[pl.ds(pl.multiple_of(kk * ZCH, 8), ZCH)])

            plsc.subcore_barrier()

            for jl in range(2):
                ts = s * 2 + jl
                pltpu.sync_copy(
                    counts_hbm.at[pl.ds(pl.multiple_of(ts * 16, 16), 16)],
                    crow)
                cv = crow[...]
                n128 = jnp.where(c == 0, cv[jq], cv[2 + jq])

                def chunk(k, _):
                    lo = pl.multiple_of((q * NW + ts) * CAP + k * GC, GC)
                    pltpu.sync_copy(bsrc_hbm.at[pl.ds(lo, GC)], srcb)
                    pltpu.sync_copy(bdstl_hbm.at[pl.ds(lo, GC)], dstlb.at[0])
                    pltpu.async_copy(h_hbm.at[srcb], rows, sem).wait()
                    pltpu.sync_copy(rows, slab.at[dstlb.at[0]], add=True)
                    return 0

                lax.fori_loop(0, n128, chunk, 0)

            plsc.subcore_barrier()
            for k in range(8):
                kk = s + k * 16

                @pl.when(kk < Q // ZCH)
                def _():
                    pltpu.sync_copy(
                        slab.at[pl.ds(pl.multiple_of(kk * ZCH, 8), ZCH)],
                        agg_hbm.at[pl.ds(pl.multiple_of(q * Q + kk * ZCH, 8),
                                         ZCH)])

            plsc.subcore_barrier()

    return kern


_aggregate = {d: _make_aggregate(d) for d in (32, 64)}


def _layer_a(h, agg, Wa, ba):
    n, din = h.shape
    dout = Wa.shape[1]
    R = 1000

    def body(h_ref, a_ref, w_ref, b_ref, y_ref, st_ref):
        i = pl.program_id(0)
        z = jnp.dot(h_ref[...] + a_ref[...], w_ref[...],
                    preferred_element_type=jnp.float32) + b_ref[...]
        y_ref[...] = z
        sums = jnp.concatenate(
            [jnp.sum(z, axis=0, keepdims=True),
             jnp.sum(z * z, axis=0, keepdims=True)], axis=0)
        st_ref[...] = jnp.where(i == 0, sums, st_ref[...] + sums)

    return pl.pallas_call(
        body,
        grid=(n // R,),
        in_specs=[
            pl.BlockSpec((R, din), lambda i: (i, 0)),
            pl.BlockSpec((R, din), lambda i: (i, 0)),
            pl.BlockSpec((din, dout), lambda i: (0, 0)),
            pl.BlockSpec((1, dout), lambda i: (0, 0)),
        ],
        out_specs=[
            pl.BlockSpec((R, dout), lambda i: (i, 0)),
            pl.BlockSpec((2, dout), lambda i: (0, 0)),
        ],
        out_shape=[
            jax.ShapeDtypeStruct((n, dout), jnp.float32),
            jax.ShapeDtypeStruct((2, dout), jnp.float32),
        ],
    )(h, agg, Wa, ba.reshape(1, dout))


def _layer_b(y, st, g, be, Wb, bb):
    n, d = y.shape
    R = 1000

    def body(y_ref, st_ref, g_ref, be_ref, w_ref, b_ref, o_ref):
        stv = st_ref[...]
        mean = stv[0:1, :] * (1.0 / n)
        var = stv[1:2, :] * (1.0 / n) - mean * mean
        inv = g_ref[...] * lax.rsqrt(var + 1e-5)
        t = be_ref[...] - mean * inv
        z = jnp.maximum(y_ref[...] * inv + t, 0.0)
        o_ref[...] = jnp.maximum(
            jnp.dot(z, w_ref[...], preferred_element_type=jnp.float32)
            + b_ref[...], 0.0)

    return pl.pallas_call(
        body,
        grid=(n // R,),
        in_specs=[
            pl.BlockSpec((R, d), lambda i: (i, 0)),
            pl.BlockSpec((2, d), lambda i: (0, 0)),
            pl.BlockSpec((1, d), lambda i: (0, 0)),
            pl.BlockSpec((1, d), lambda i: (0, 0)),
            pl.BlockSpec((d, d), lambda i: (0, 0)),
            pl.BlockSpec((1, d), lambda i: (0, 0)),
        ],
        out_specs=pl.BlockSpec((R, d), lambda i: (i, 0)),
        out_shape=jax.ShapeDtypeStruct((n, d), jnp.float32),
    )(y, st, g.reshape(1, d), be.reshape(1, d), Wb, bb.reshape(1, d))


def _pool_nodes(h3, orders):
    HALF = N // 2
    NCH = HALF // 128

    @functools.partial(
        pl.kernel,
        out_type=jax.ShapeDtypeStruct((2, G, 64), jnp.float32),
        mesh=_mesh(),
        compiler_params=pltpu.CompilerParams(
            needs_layout_passes=False, use_tc_tiling_on_sc=False),
        scratch_types=[
            pltpu.VMEM_SHARED((G, 64), jnp.float32),
            pltpu.VMEM((128, 64), jnp.float32),
            pltpu.VMEM((1, 128), jnp.int32),
            pltpu.VMEM((128, 64), jnp.float32),
            pltpu.VMEM((1, 80), jnp.int32),
            pltpu.VMEM((80, 64), jnp.float32),
        ],
    )
    def kern(h_hbm, ord_hbm, out_hbm, slab, zbuf, ordrow, rowsb, ordt, rowst):
        c = lax.axis_index("c")
        s = lax.axis_index("s")
        zv = jnp.zeros((16,), jnp.float32)

        def zrow(i, _):
            for j in range(4):
                zbuf[i, pl.ds(j * 16, 16)] = zv
            return 0

        lax.fori_loop(0, 128, zrow, 0)

        @pl.when(s < 15)
        def _():
            pltpu.sync_copy(
                zbuf, slab.at[pl.ds(pl.multiple_of(s * 128, 128), 128)])

        @pl.when(s == 15)
        def _():
            pltpu.sync_copy(zbuf.at[pl.ds(0, 80)], slab.at[pl.ds(1920, 80)])

        plsc.subcore_barrier()

        def chunk(k, _):
            kk = s + k * 16

            @pl.when(kk < NCH)
            def _():
                base = pl.multiple_of(c * HALF + kk * 128, 8)
                pltpu.sync_copy(ord_hbm.at[pl.ds(base, 128)], ordrow.at[0])
                pltpu.sync_copy(h_hbm.at[pl.ds(base, 128)], rowsb)
                pltpu.sync_copy(rowsb, slab.at[ordrow.at[0]], add=True)

            return 0

        lax.fori_loop(0, (NCH + 15) // 16, chunk, 0)

        @pl.when(s == 15)
        def _():
            base = pl.multiple_of(c * HALF + NCH * 128, 8)
            pltpu.sync_copy(ord_hbm.at[pl.ds(base, 80)], ordt.at[0])
            pltpu.sync_copy(h_hbm.at[pl.ds(base, 80)], rowst)
            pltpu.sync_copy(rowst, slab.at[ordt.at[0]], add=True)

        plsc.subcore_barrier()

        @pl.when(s < 15)
        def _():
            so = pl.multiple_of(s * 128, 128)
            pltpu.sync_copy(slab.at[pl.ds(so, 128)],
                            out_hbm.at[c, pl.ds(so, 128)])

        @pl.when(s == 15)
        def _():
            pltpu.sync_copy(slab.at[pl.ds(1920, 80)],
                            out_hbm.at[c, pl.ds(1920, 80)])

    return kern(h3, orders)


def _pool_graphs(hg, orders):
    HALF = G // 2

    @functools.partial(
        pl.kernel,
        out_type=jax.ShapeDtypeStruct((2, B, 128), jnp.float32),
        mesh=_mesh(),
        compiler_params=pltpu.CompilerParams(
            needs_layout_passes=False, use_tc_tiling_on_sc=False),
        scratch_types=[
            pltpu.VMEM_SHARED((B, 128), jnp.float32),
            pltpu.VMEM((128, 128), jnp.float32),
            pltpu.VMEM((1, 128), jnp.int32),
            pltpu.VMEM((128, 128), jnp.float32),
            pltpu.VMEM((1, 104), jnp.int32),
            pltpu.VMEM((104, 128), jnp.float32),
        ],
    )
    def kern(h_hbm, ord_hbm, out_hbm, slab, zbuf, ordrow, rowsb, ordt, rowst):
        c = lax.axis_index("c")
        s = lax.axis_index("s")
        zv = jnp.zeros((16,), jnp.float32)

        def zrow(i, _):
            for j in range(8):
                zbuf[i, pl.ds(j * 16, 16)] = zv
            return 0

        lax.fori_loop(0, 128, zrow, 0)

        @pl.when(s < 7)
        def _():
            pltpu.sync_copy(
                zbuf, slab.at[pl.ds(pl.multiple_of(s * 128, 128), 128)])

        @pl.when(s == 7)
        def _():
            pltpu.sync_copy(zbuf.at[pl.ds(0, 104)], slab.at[pl.ds(896, 104)])

        plsc.subcore_barrier()

        @pl.when(s < 7)
        def _():
            base = pl.multiple_of(c * HALF + s * 128, 8)
            pltpu.sync_copy(ord_hbm.at[pl.ds(base, 128)], ordrow.at[0])
            pltpu.sync_copy(h_hbm.at[pl.ds(base, 128)], rowsb)
            pltpu.sync_copy(rowsb, slab.at[ordrow.at[0]], add=True)

        @pl.when(s == 7)
        def _():
            base = pl.multiple_of(c * HALF + 896, 8)
            pltpu.sync_copy(ord_hbm.at[pl.ds(base, 104)], ordt.at[0])
            pltpu.sync_copy(h_hbm.at[pl.ds(base, 104)], rowst)
            pltpu.sync_copy(rowst, slab.at[ordt.at[0]], add=True)

        plsc.subcore_barrier()

        @pl.when(s < 7)
        def _():
            so = pl.multiple_of(s * 128, 128)
            pltpu.sync_copy(slab.at[pl.ds(so, 128)],
                            out_hbm.at[c, pl.ds(so, 128)])

        @pl.when(s == 7)
        def _():
            pltpu.sync_copy(slab.at[pl.ds(896, 104)],
                            out_hbm.at[c, pl.ds(896, 104)])

    return kern(hg, orders)


def _head1(gp, Wl1, bl1):
    def body(p_ref, w_ref, b_ref, o_ref):
        ph = p_ref[0] + p_ref[1]
        o_ref[...] = jnp.maximum(
            jnp.dot(ph, w_ref[...], preferred_element_type=jnp.float32)
            + b_ref[...], 0.0)

    return pl.pallas_call(
        body,
        out_shape=jax.ShapeDtypeStruct((G, 128), jnp.float32),
    )(gp, Wl1, bl1.reshape(1, 128))


def _head2(bp, Wl2, bl2):
    def body(p_ref, w_ref, b_ref, o_ref):
        ph = p_ref[0] + p_ref[1]
        logits = jnp.sum(ph * w_ref[...], axis=1, keepdims=True) + b_ref[...]
        o_ref[...] = 1.0 / (1.0 + jnp.exp(-logits))

    return pl.pallas_call(
        body,
        out_shape=jax.ShapeDtypeStruct((B, 1), jnp.float32),
    )(bp, Wl2.reshape(1, 128), bl2.reshape(1, 1))


def kernel(x, edge_index, nodes_orders, batch_orders,
           W1a, b1a, g1, be1, W1b, b1b,
           W2a, b2a, g2, be2, W2b, b2b,
           W3a, b3a, g3, be3, W3b, b3b,
           Wl1, bl1, Wl2, bl2):
    src_e = edge_index[0].astype(jnp.int32)
    dst_e = edge_index[1].astype(jnp.int32)
    x32 = jnp.pad(x, ((0, 0), (0, 2)))
    W1a32 = jnp.pad(W1a, ((0, 2), (0, 0)))

    bsrc, bdstl, counts = _bucketize(src_e, dst_e)

    h = x32
    for Wa, ba, g, be, Wb, bb in (
        (W1a32, b1a, g1, be1, W1b, b1b),
        (W2a, b2a, g2, be2, W2b, b2b),
        (W3a, b3a, g3, be3, W3b, b3b),
    ):
        agg = _aggregate[h.shape[1]](h, bsrc, bdstl, counts)
        y, st = _layer_a(h, agg, Wa, ba)
        h = _layer_b(y, st, g, be, Wb, bb)

    gp = _pool_nodes(h, nodes_orders.astype(jnp.int32))
    hg = _head1(gp, Wl1, bl1)
    bp = _pool_graphs(hg, batch_orders.astype(jnp.int32))
    out = _head2(bp, Wl2, bl2)
    return out[:, 0]


# speedup vs baseline: 8.0839x; 1.4491x over previous
import functools

import jax
import jax.numpy as jnp
from jax import lax
from jax.experimental import pallas as pl
from jax.experimental.pallas import tpu as pltpu
from jax.experimental.pallas import tpu_sc as plsc

N = 100000
E = 1600000
G = 2000
B = 1000

NQ = 4
Q = 25000
ZCH = 200
SLAB_ROWS = 25216
NW = 32
CAP = 20480
ECH = 2000
STG = 4096
FLUSH = 2048
GC = 128


def _mesh():
    return plsc.VectorSubcoreMesh(core_axis_name="c", subcore_axis_name="s")


def _bucketize(src_e, dst_e):
    @functools.partial(
        pl.kernel,
        out_type=[
            jax.ShapeDtypeStruct((NQ * NW * CAP,), jnp.int32),
            jax.ShapeDtypeStruct((NQ * NW * CAP,), jnp.int32),
            jax.ShapeDtypeStruct((NW * 16,), jnp.int32),
        ],
        mesh=_mesh(),
        compiler_params=pltpu.CompilerParams(
            needs_layout_passes=False, use_tc_tiling_on_sc=False),
        scratch_types=[
            pltpu.VMEM((ECH,), jnp.int32),
            pltpu.VMEM((ECH,), jnp.int32),
            pltpu.VMEM((NQ * STG,), jnp.int32),
            pltpu.VMEM((NQ * STG,), jnp.int32),
            pltpu.VMEM((16,), jnp.int32),
        ],
    )
    def kern(src_hbm, dst_hbm, bsrc_hbm, bdstl_hbm, counts_hbm,
             estg0, estg1, sstg, dstg, crow):
        c = lax.axis_index("c")
        s = lax.axis_index("s")
        w = c * 16 + s
        base = w * (E // NW)
        lane = lax.iota(jnp.int32, 16)
        zeros16 = jnp.zeros((16,), jnp.int32)
        trash16 = jnp.zeros((16,), jnp.int32) + (Q + w)

        def outer(i, carry):
            eo = pl.multiple_of(base + i * ECH, 2000)
            pltpu.sync_copy(src_hbm.at[pl.ds(eo, ECH)], estg0)
            pltpu.sync_copy(dst_hbm.at[pl.ds(eo, ECH)], estg1)

            def vloop(v, carry2):
                offs, fls = carry2
                src16 = estg0[pl.ds(v * 16, 16)]
                dst16 = estg1[pl.ds(v * 16, 16)]
                q16 = (
                    (dst16 >= Q).astype(jnp.int32)
                    + (dst16 >= 2 * Q).astype(jnp.int32)
                    + (dst16 >= 3 * Q).astype(jnp.int32)
                )
                new_offs = []
                new_fls = []
                for b in range(NQ):
                    off_b = offs[b]
                    fl_b = fls[b]
                    m = q16 == b
                    cs = plsc.cumsum(m.astype(jnp.int32))
                    idx = b * STG + off_b + cs - 1
                    plsc.store_scatter(sstg, [idx], src16, mask=m)
                    plsc.store_scatter(dstg, [idx], dst16 - b * Q, mask=m)
                    off_b = off_b + cs[15]
                    do_flush = off_b >= FLUSH

                    @pl.when(do_flush)
                    def _():
                        fo = pl.multiple_of(
                            (b * NW + w) * CAP + fl_b, 2048)
                        pltpu.sync_copy(
                            sstg.at[pl.ds(b * STG, FLUSH)],
                            bsrc_hbm.at[pl.ds(fo, FLUSH)])
                        pltpu.sync_copy(
                            dstg.at[pl.ds(b * STG, FLUSH)],
                            bdstl_hbm.at[pl.ds(fo, FLUSH)])
                        tail_s = sstg[pl.ds(b * STG + FLUSH, 16)]
                        tail_d = dstg[pl.ds(b * STG + FLUSH, 16)]
                        sstg[pl.ds(b * STG, 16)] = tail_s
                        dstg[pl.ds(b * STG, 16)] = tail_d

                    new_offs.append(jnp.where(do_flush, off_b - FLUSH, off_b))
                    new_fls.append(jnp.where(do_flush, fl_b + FLUSH, fl_b))
                return (new_offs, new_fls)

            return lax.fori_loop(0, ECH // 16, vloop, carry)

        zero = jnp.int32(0)
        offs, fls = lax.fori_loop(
            0, (E // NW) // ECH, outer,
            ([zero] * NQ, [zero] * NQ))

        crow_v = jnp.zeros((16,), jnp.int32)
        for b in range(NQ):
            off_b = offs[b]
            fl_b = fls[b]
            total = fl_b + off_b
            padded = ((total + 127) >> 7) << 7
            for j in range(16):
                pidx = b * STG + off_b + j * 16 + lane
                plsc.store_scatter(sstg, [pidx], zeros16)
                plsc.store_scatter(dstg, [pidx], trash16)
            fo = pl.multiple_of((b * NW + w) * CAP + fl_b, 2048)
            pltpu.sync_copy(sstg.at[pl.ds(b * STG, STG)],
                            bsrc_hbm.at[pl.ds(fo, STG)])
            pltpu.sync_copy(dstg.at[pl.ds(b * STG, STG)],
                            bdstl_hbm.at[pl.ds(fo, STG)])
            crow_v = jnp.where(lane == b, padded >> 7, crow_v)
        crow[...] = crow_v
        pltpu.sync_copy(crow, counts_hbm.at[pl.ds(pl.multiple_of(w * 16, 16),
                                                  16)])

    return kern(src_e, dst_e)


BLK = 2048


def _make_aggregate(D):
    @functools.partial(
        pl.kernel,
        out_type=jax.ShapeDtypeStruct((N, D), jnp.float32),
        mesh=_mesh(),
        compiler_params=pltpu.CompilerParams(
            needs_layout_passes=False, use_tc_tiling_on_sc=False),
        scratch_types=[
            pltpu.VMEM_SHARED((SLAB_ROWS, D), jnp.float32),
            pltpu.VMEM((BLK,), jnp.int32),
            pltpu.VMEM((BLK,), jnp.int32),
            pltpu.VMEM((BLK,), jnp.int32),
            pltpu.VMEM((BLK,), jnp.int32),
            pltpu.VMEM((GC, D), jnp.float32),
            pltpu.VMEM((GC, D), jnp.float32),
            pltpu.VMEM((16,), jnp.int32),
            pltpu.SemaphoreType.DMA,
            pltpu.SemaphoreType.DMA,
        ],
    )
    def kern(h_hbm, bsrc_hbm, bdstl_hbm, counts_hbm, agg_hbm,
             slab, srcA, dstA, srcB, dstB, rows0, rows1, crow, sem0, sem1):
        c = lax.axis_index("c")
        s = lax.axis_index("s")
        zv = jnp.zeros((16,), jnp.float32)

        def zero_rows0(i, _):
            for j in range(D // 16):
                rows0[i, pl.ds(j * 16, 16)] = zv
            return 0

        def gwait(buf, sem):
            pltpu.make_async_copy(h_hbm.at[pl.ds(0, GC)], buf, sem).wait()

        for jq in range(2):
            q = 2 * c + jq
            lax.fori_loop(0, GC, zero_rows0, 0)
            for k in range(13):
                kk = s + k * 16

                @pl.when(kk < SLAB_ROWS // GC)
                def _():
                    pltpu.sync_copy(
                        rows0,
                        slab.at[pl.ds(pl.multiple_of(kk * GC, 8), GC)])

            plsc.subcore_barrier()

            for jl in range(2):
                ts = s * 2 + jl
                pltpu.sync_copy(
                    counts_hbm.at[pl.ds(pl.multiple_of(ts * 16, 16), 16)],
                    crow)
                cv = crow[...]
                n128 = jnp.where(c == 0, cv[jq], cv[2 + jq])
                lo = pl.multiple_of((q * NW + ts) * CAP, GC)
                nc = ((n128 + 1) >> 1) << 1

                def load_blk(bidx):
                    bo = pl.multiple_of(lo + bidx * BLK, BLK)

                    @pl.when((bidx & 1) == 0)
                    def _():
                        pltpu.sync_copy(bsrc_hbm.at[pl.ds(bo, BLK)], srcA)
                        pltpu.sync_copy(bdstl_hbm.at[pl.ds(bo, BLK)], dstA)

                    @pl.when((bidx & 1) == 1)
                    def _():
                        pltpu.sync_copy(bsrc_hbm.at[pl.ds(bo, BLK)], srcB)
                        pltpu.sync_copy(bdstl_hbm.at[pl.ds(bo, BLK)], dstB)

                def gstart(k, buf, sem):
                    ko = pl.multiple_of((k & 15) * GC, GC)
                    par = (k >> 4) & 1

                    @pl.when(par == 0)
                    def _():
                        pltpu.async_copy(
                            h_hbm.at[srcA.at[pl.ds(ko, GC)]], buf, sem)

                    @pl.when(par == 1)
                    def _():
                        pltpu.async_copy(
                            h_hbm.at[srcB.at[pl.ds(ko, GC)]], buf, sem)

                def scat(k, buf):
                    ko = pl.multiple_of((k & 15) * GC, GC)
                    par = (k >> 4) & 1

                    @pl.when(par == 0)
                    def _():
                        pltpu.sync_copy(
                            buf, slab.at[dstA.at[pl.ds(ko, GC)]], add=True)

                    @pl.when(par == 1)
                    def _():
                        pltpu.sync_copy(
                            buf, slab.at[dstB.at[pl.ds(ko, GC)]], add=True)

                @pl.when(nc > 0)
                def _():
                    load_blk(jnp.int32(0))
                    gstart(jnp.int32(0), rows0, sem0)
                    gstart(jnp.int32(1), rows1, sem1)

                    def body(p, _):
                        k0 = 2 * p
                        gwait(rows0, sem0)
                        scat(k0, rows0)

                        @pl.when(k0 + 2 < nc)
                        def _():
                            @pl.when(((k0 + 2) & 15) == 0)
                            def _():
                                load_blk((k0 + 2) >> 4)

                            gstart(k0 + 2, rows0, sem0)

                        gwait(rows1, sem1)
                        scat(k0 + 1, rows1)

                        @pl.when(k0 + 3 < nc)
                        def _():
                            gstart(k0 + 3, rows1, sem1)

                        return 0

                    lax.fori_loop(0, nc >> 1, body, 0)

            plsc.subcore_barrier()
            for k in range(8):
                kk = s + k * 16

                @pl.when(kk < Q // ZCH)
                def _():
                    pltpu.sync_copy(
                        slab.at[pl.ds(pl.multiple_of(kk * ZCH, 8), ZCH)],
                        agg_hbm.at[pl.ds(pl.multiple_of(q * Q + kk * ZCH, 8),
                                         ZCH)])

            plsc.subcore_barrier()

    return kern


_aggregate = {d: _make_aggregate(d) for d in (32, 64)}


def _layer_a(h, agg, Wa, ba):
    n, din = h.shape
    dout = Wa.shape[1]
    R = 1000

    def body(h_ref, a_ref, w_ref, b_ref, y_ref, st_ref):
        i = pl.program_id(0)
        z = jnp.dot(h_ref[...] + a_ref[...], w_ref[...],
                    preferred_element_type=jnp.float32) + b_ref[...]
        y_ref[...] = z
        sums = jnp.concatenate(
            [jnp.sum(z, axis=0, keepdims=True),
             jnp.sum(z * z, axis=0, keepdims=True)], axis=0)
        st_ref[...] = jnp.where(i == 0, sums, st_ref[...] + sums)

    return pl.pallas_call(
        body,
        grid=(n // R,),
        in_specs=[
            pl.BlockSpec((R, din), lambda i: (i, 0)),
            pl.BlockSpec((R, din), lambda i: (i, 0)),
            pl.BlockSpec((din, dout), lambda i: (0, 0)),
            pl.BlockSpec((1, dout), lambda i: (0, 0)),
        ],
        out_specs=[
            pl.BlockSpec((R, dout), lambda i: (i, 0)),
            pl.BlockSpec((2, dout), lambda i: (0, 0)),
        ],
        out_shape=[
            jax.ShapeDtypeStruct((n, dout), jnp.float32),
            jax.ShapeDtypeStruct((2, dout), jnp.float32),
        ],
    )(h, agg, Wa, ba.reshape(1, dout))


def _layer_b(y, st, g, be, Wb, bb):
    n, d = y.shape
    R = 1000

    def body(y_ref, st_ref, g_ref, be_ref, w_ref, b_ref, o_ref):
        stv = st_ref[...]
        mean = stv[0:1, :] * (1.0 / n)
        var = stv[1:2, :] * (1.0 / n) - mean * mean
        inv = g_ref[...] * lax.rsqrt(var + 1e-5)
        t = be_ref[...] - mean * inv
        z = jnp.maximum(y_ref[...] * inv + t, 0.0)
        o_ref[...] = jnp.maximum(
            jnp.dot(z, w_ref[...], preferred_element_type=jnp.float32)
            + b_ref[...], 0.0)

    return pl.pallas_call(
        body,
        grid=(n // R,),
        in_specs=[
            pl.BlockSpec((R, d), lambda i: (i, 0)),
            pl.BlockSpec((2, d), lambda i: (0, 0)),
            pl.BlockSpec((1, d), lambda i: (0, 0)),
            pl.BlockSpec((1, d), lambda i: (0, 0)),
            pl.BlockSpec((d, d), lambda i: (0, 0)),
            pl.BlockSpec((1, d), lambda i: (0, 0)),
        ],
        out_specs=pl.BlockSpec((R, d), lambda i: (i, 0)),
        out_shape=jax.ShapeDtypeStruct((n, d), jnp.float32),
    )(y, st, g.reshape(1, d), be.reshape(1, d), Wb, bb.reshape(1, d))


def _pool_nodes(h3, orders):
    HALF = N // 2
    NCH = HALF // 128

    @functools.partial(
        pl.kernel,
        out_type=jax.ShapeDtypeStruct((2, G, 64), jnp.float32),
        mesh=_mesh(),
        compiler_params=pltpu.CompilerParams(
            needs_layout_passes=False, use_tc_tiling_on_sc=False),
        scratch_types=[
            pltpu.VMEM_SHARED((G, 64), jnp.float32),
            pltpu.VMEM((128, 64), jnp.float32),
            pltpu.VMEM((1, 128), jnp.int32),
            pltpu.VMEM((128, 64), jnp.float32),
            pltpu.VMEM((1, 80), jnp.int32),
            pltpu.VMEM((80, 64), jnp.float32),
        ],
    )
    def kern(h_hbm, ord_hbm, out_hbm, slab, zbuf, ordrow, rowsb, ordt, rowst):
        c = lax.axis_index("c")
        s = lax.axis_index("s")
        zv = jnp.zeros((16,), jnp.float32)

        def zrow(i, _):
            for j in range(4):
                zbuf[i, pl.ds(j * 16, 16)] = zv
            return 0

        lax.fori_loop(0, 128, zrow, 0)

        @pl.when(s < 15)
        def _():
            pltpu.sync_copy(
                zbuf, slab.at[pl.ds(pl.multiple_of(s * 128, 128), 128)])

        @pl.when(s == 15)
        def _():
            pltpu.sync_copy(zbuf.at[pl.ds(0, 80)], slab.at[pl.ds(1920, 80)])

        plsc.subcore_barrier()

        def chunk(k, _):
            kk = s + k * 16

            @pl.when(kk < NCH)
            def _():
                base = pl.multiple_of(c * HALF + kk * 128, 8)
                pltpu.sync_copy(ord_hbm.at[pl.ds(base, 128)], ordrow.at[0])
                pltpu.sync_copy(h_hbm.at[pl.ds(base, 128)], rowsb)
                pltpu.sync_copy(rowsb, slab.at[ordrow.at[0]], add=True)

            return 0

        lax.fori_loop(0, (NCH + 15) // 16, chunk, 0)

        @pl.when(s == 15)
        def _():
            base = pl.multiple_of(c * HALF + NCH * 128, 8)
            pltpu.sync_copy(ord_hbm.at[pl.ds(base, 80)], ordt.at[0])
            pltpu.sync_copy(h_hbm.at[pl.ds(base, 80)], rowst)
            pltpu.sync_copy(rowst, slab.at[ordt.at[0]], add=True)

        plsc.subcore_barrier()

        @pl.when(s < 15)
        def _():
            so = pl.multiple_of(s * 128, 128)
            pltpu.sync_copy(slab.at[pl.ds(so, 128)],
                            out_hbm.at[c, pl.ds(so, 128)])

        @pl.when(s == 15)
        def _():
            pltpu.sync_copy(slab.at[pl.ds(1920, 80)],
                            out_hbm.at[c, pl.ds(1920, 80)])

    return kern(h3, orders)


def _pool_graphs(hg, orders):
    HALF = G // 2

    @functools.partial(
        pl.kernel,
        out_type=jax.ShapeDtypeStruct((2, B, 128), jnp.float32),
        mesh=_mesh(),
        compiler_params=pltpu.CompilerParams(
            needs_layout_passes=False, use_tc_tiling_on_sc=False),
        scratch_types=[
            pltpu.VMEM_SHARED((B, 128), jnp.float32),
            pltpu.VMEM((128, 128), jnp.float32),
            pltpu.VMEM((1, 128), jnp.int32),
            pltpu.VMEM((128, 128), jnp.float32),
            pltpu.VMEM((1, 104), jnp.int32),
            pltpu.VMEM((104, 128), jnp.float32),
        ],
    )
    def kern(h_hbm, ord_hbm, out_hbm, slab, zbuf, ordrow, rowsb, ordt, rowst):
        c = lax.axis_index("c")
        s = lax.axis_index("s")
        zv = jnp.zeros((16,), jnp.float32)

        def zrow(i, _):
            for j in range(8):
                zbuf[i, pl.ds(j * 16, 16)] = zv
            return 0

        lax.fori_loop(0, 128, zrow, 0)

        @pl.when(s < 7)
        def _():
            pltpu.sync_copy(
                zbuf, slab.at[pl.ds(pl.multiple_of(s * 128, 128), 128)])

        @pl.when(s == 7)
        def _():
            pltpu.sync_copy(zbuf.at[pl.ds(0, 104)], slab.at[pl.ds(896, 104)])

        plsc.subcore_barrier()

        @pl.when(s < 7)
        def _():
            base = pl.multiple_of(c * HALF + s * 128, 8)
            pltpu.sync_copy(ord_hbm.at[pl.ds(base, 128)], ordrow.at[0])
            pltpu.sync_copy(h_hbm.at[pl.ds(base, 128)], rowsb)
            pltpu.sync_copy(rowsb, slab.at[ordrow.at[0]], add=True)

        @pl.when(s == 7)
        def _():
            base = pl.multiple_of(c * HALF + 896, 8)
            pltpu.sync_copy(ord_hbm.at[pl.ds(base, 104)], ordt.at[0])
            pltpu.sync_copy(h_hbm.at[pl.ds(base, 104)], rowst)
            pltpu.sync_copy(rowst, slab.at[ordt.at[0]], add=True)

        plsc.subcore_barrier()

        @pl.when(s < 7)
        def _():
            so = pl.multiple_of(s * 128, 128)
            pltpu.sync_copy(slab.at[pl.ds(so, 128)],
                            out_hbm.at[c, pl.ds(so, 128)])

        @pl.when(s == 7)
        def _():
            pltpu.sync_copy(slab.at[pl.ds(896, 104)],
                            out_hbm.at[c, pl.ds(896, 104)])

    return kern(hg, orders)


def _head1(gp, Wl1, bl1):
    def body(p_ref, w_ref, b_ref, o_ref):
        ph = p_ref[0] + p_ref[1]
        o_ref[...] = jnp.maximum(
            jnp.dot(ph, w_ref[...], preferred_element_type=jnp.float32)
            + b_ref[...], 0.0)

    return pl.pallas_call(
        body,
        out_shape=jax.ShapeDtypeStruct((G, 128), jnp.float32),
    )(gp, Wl1, bl1.reshape(1, 128))


def _head2(bp, Wl2, bl2):
    def body(p_ref, w_ref, b_ref, o_ref):
        ph = p_ref[0] + p_ref[1]
        logits = jnp.sum(ph * w_ref[...], axis=1, keepdims=True) + b_ref[...]
        o_ref[...] = 1.0 / (1.0 + jnp.exp(-logits))

    return pl.pallas_call(
        body,
        out_shape=jax.ShapeDtypeStruct((B, 1), jnp.float32),
    )(bp, Wl2.reshape(1, 128), bl2.reshape(1, 1))


def kernel(x, edge_index, nodes_orders, batch_orders,
           W1a, b1a, g1, be1, W1b, b1b,
           W2a, b2a, g2, be2, W2b, b2b,
           W3a, b3a, g3, be3, W3b, b3b,
           Wl1, bl1, Wl2, bl2):
    src_e = edge_index[0].astype(jnp.int32)
    dst_e = edge_index[1].astype(jnp.int32)
    x32 = jnp.pad(x, ((0, 0), (0, 2)))
    W1a32 = jnp.pad(W1a, ((0, 2), (0, 0)))

    bsrc, bdstl, counts = _bucketize(src_e, dst_e)

    h = x32
    for Wa, ba, g, be, Wb, bb in (
        (W1a32, b1a, g1, be1, W1b, b1b),
        (W2a, b2a, g2, be2, W2b, b2b),
        (W3a, b3a, g3, be3, W3b, b3b),
    ):
        agg = _aggregate[h.shape[1]](h, bsrc, bdstl, counts)
        y, st = _layer_a(h, agg, Wa, ba)
        h = _layer_b(y, st, g, be, Wb, bb)

    gp = _pool_nodes(h, nodes_orders.astype(jnp.int32))
    hg = _head1(gp, Wl1, bl1)
    bp = _pool_graphs(hg, batch_orders.astype(jnp.int32))
    out = _head2(bp, Wl2, bl2)
    return out[:, 0]
